# baseline (speedup 1.0000x reference)
import functools

import jax
import jax.numpy as jnp
from jax import lax
from jax.experimental import pallas as pl
from jax.experimental.pallas import tpu as pltpu
from jax.experimental.pallas import tpu_sc as plsc

N = 10000
E = 320000
F = 128

NC = 2
NS = 16
NW = NC * NS

CHUNK = 128
NCHUNKS_PAD = 2560
E_PAD = NCHUNKS_PAD * CHUNK
CH_W = NCHUNKS_PAD // NW
IB = 40
NIB = CH_W // IB
N_PAD = N + 8
SUB_ROWS = 632

_PREC = lax.Precision.HIGHEST


NBUF = 2


def _sc_body(with_deg, *refs):
    if with_deg:
        (src_hbm, dst_hbm, p_hbm, zf_hbm, z1_hbm,
         agg_hbm, deg_hbm, src_ib, dst_ib,
         rv0, rv1, ones_v, dbuf_v,
         agg_sh, deg_sh,
         g0, g1, s0, s1, dsem) = refs
    else:
        (src_hbm, dst_hbm, p_hbm, zf_hbm,
         agg_hbm, src_ib, dst_ib,
         rv0, rv1, agg_sh,
         g0, g1, s0, s1) = refs
    bufs = (rv0, rv1)
    gsems = (g0, g1)
    ssems = (s0, s1)

    c = lax.axis_index("c")
    s = lax.axis_index("s")
    w = s * NC + c
    rbase = pl.multiple_of(
        jnp.where(s < NS - 1, s * SUB_ROWS, N_PAD - SUB_ROWS), 8)

    pltpu.sync_copy(zf_hbm, agg_sh.at[pl.ds(rbase, SUB_ROWS)])
    if with_deg:
        pltpu.sync_copy(z1_hbm, dbuf_v)
        pltpu.sync_copy(dbuf_v, deg_sh.at[pl.ds(rbase, SUB_ROWS)])

        @pl.loop(0, CHUNK // 16)
        def _(i):
            ones_v[pl.ds(i * 16, 16)] = jnp.ones((16,), jnp.float32)
    plsc.subcore_barrier()

    def gather(i, b):
        pltpu.async_copy(p_hbm.at[src_ib.at[i]], bufs[b], gsems[b])

    def wait_gather(i, b):
        pltpu.make_async_copy(p_hbm.at[src_ib.at[i]], bufs[b], gsems[b]).wait()

    def scatter(i, b):
        pltpu.async_copy(bufs[b], agg_sh.at[dst_ib.at[i]], ssems[b], add=True)
        if with_deg:
            pltpu.async_copy(ones_v, deg_sh.at[dst_ib.at[i]], dsem, add=True)

    def wait_scatter(i, b):
        pltpu.make_async_copy(bufs[b], agg_sh.at[dst_ib.at[i]],
                              ssems[b]).wait()

    @pl.loop(0, NIB)
    def _(blk):
        crow = pl.multiple_of((w * NIB + blk) * IB, 8)
        pltpu.sync_copy(src_hbm.at[pl.ds(crow, IB)], src_ib)
        pltpu.sync_copy(dst_hbm.at[pl.ds(crow, IB)], dst_ib)
        gather(0, 0)
        for j in range(IB):
            if j + 1 < IB:
                if j >= 1:
                    wait_scatter(j - 1, (j + 1) % NBUF)
                gather(j + 1, (j + 1) % NBUF)
            wait_gather(j, j % NBUF)
            scatter(j, j % NBUF)
        wait_scatter(IB - 1, (IB - 1) % NBUF)
        if with_deg:
            for j in range(IB):
                pltpu.make_async_copy(ones_v, deg_sh.at[dst_ib.at[j]],
                                      dsem).wait()

    plsc.subcore_barrier()

    pltpu.sync_copy(agg_sh.at[pl.ds(rbase, SUB_ROWS)],
                    agg_hbm.at[c, pl.ds(rbase, SUB_ROWS)])
    if with_deg:
        dbase = pl.multiple_of(c * N_PAD + rbase, 8)
        pltpu.sync_copy(deg_sh.at[pl.ds(rbase, SUB_ROWS)], dbuf_v)
        pltpu.sync_copy(dbuf_v, deg_hbm.at[pl.ds(dbase, SUB_ROWS)])


def _make_sc_pass(with_deg):
    mesh = plsc.VectorSubcoreMesh(core_axis_name="c", subcore_axis_name="s",
                                  num_cores=NC, num_subcores=NS)
    out_type = [jax.ShapeDtypeStruct((NC, N_PAD, F), jnp.float32)]
    scratch = [
        pltpu.VMEM((IB, CHUNK), jnp.int32),
        pltpu.VMEM((IB, CHUNK), jnp.int32),
    ]
    scratch += [pltpu.VMEM((CHUNK, F), jnp.float32)] * NBUF

    if with_deg:
        out_type.append(jax.ShapeDtypeStruct((NC * N_PAD,), jnp.float32))
        scratch.append(pltpu.VMEM((CHUNK,), jnp.float32))
        scratch.append(pltpu.VMEM((SUB_ROWS,), jnp.float32))
    scratch.append(pltpu.VMEM_SHARED((N_PAD, F), jnp.float32))
    if with_deg:
        scratch.append(pltpu.VMEM_SHARED((N_PAD,), jnp.float32))
    scratch += [pltpu.SemaphoreType.DMA] * (2 * NBUF + (1 if with_deg else 0))

    return pl.kernel(functools.partial(_sc_body, with_deg),
                     out_type=tuple(out_type) if with_deg else out_type[0],
                     mesh=mesh, scratch_types=scratch)


BM = 1000

_ROW = pl.BlockSpec((BM, F), lambda i: (i, 0))
_ROW1 = pl.BlockSpec((BM, 1), lambda i: (i, 0))
_PART = pl.BlockSpec((NC, BM, F), lambda i: (0, i, 0))
_W = pl.BlockSpec((F, F), lambda i: (0, 0))
_B = pl.BlockSpec((1, F), lambda i: (0, 0))
_GRID = (N // BM,)


def _prep_body(x_ref, wn_ref, ws_ref, b_ref, p_ref, s_ref):
    x = x_ref[...]
    p_ref[...] = lax.dot(x, wn_ref[...], precision=_PREC)
    s_ref[...] = lax.dot(x, ws_ref[...], precision=_PREC) + b_ref[...]


def _prep(x, wn, ws, b):
    return pl.pallas_call(
        _prep_body,
        grid=_GRID,
        in_specs=[_ROW, _W, _W, _B],
        out_specs=(_ROW, _ROW),
        out_shape=(jax.ShapeDtypeStruct((N, F), jnp.float32),
                   jax.ShapeDtypeStruct((N, F), jnp.float32)),
    )(x, wn, ws, b)


def _combine1_body(s_ref, agg_ref, deg0_ref, deg1_ref, wn_ref, ws_ref, b_ref,
                   p_ref, s2_ref, invd_ref):
    deg = deg0_ref[...] + deg1_ref[...]
    invd = 1.0 / jnp.maximum(deg, 1.0)
    invd_ref[...] = invd
    agg = agg_ref[0] + agg_ref[1]
    h = jax.nn.sigmoid(s_ref[...] + agg * invd)
    p_ref[...] = lax.dot(h, wn_ref[...], precision=_PREC)
    s2_ref[...] = lax.dot(h, ws_ref[...], precision=_PREC) + b_ref[...]


def _combine1(s1, aggp, deg0, deg1, wn, ws, b):
    return pl.pallas_call(
        _combine1_body,
        grid=_GRID,
        in_specs=[_ROW, _PART, _ROW1, _ROW1, _W, _W, _B],
        out_specs=(_ROW, _ROW, _ROW1),
        out_shape=(jax.ShapeDtypeStruct((N, F), jnp.float32),
                   jax.ShapeDtypeStruct((N, F), jnp.float32),
                   jax.ShapeDtypeStruct((N, 1), jnp.float32)),
    )(s1, aggp, deg0, deg1, wn, ws, b)


def _combine_mid_body(s_ref, agg_ref, invd_ref, wn_ref, ws_ref, b_ref,
                      p_ref, s2_ref):
    agg = agg_ref[0] + agg_ref[1]
    h = jax.nn.sigmoid(s_ref[...] + agg * invd_ref[...])
    p_ref[...] = lax.dot(h, wn_ref[...], precision=_PREC)
    s2_ref[...] = lax.dot(h, ws_ref[...], precision=_PREC) + b_ref[...]


def _combine_mid(s_in, aggp, invd, wn, ws, b):
    return pl.pallas_call(
        _combine_mid_body,
        grid=_GRID,
        in_specs=[_ROW, _PART, _ROW1, _W, _W, _B],
        out_specs=(_ROW, _ROW),
        out_shape=(jax.ShapeDtypeStruct((N, F), jnp.float32),
                   jax.ShapeDtypeStruct((N, F), jnp.float32)),
    )(s_in, aggp, invd, wn, ws, b)


def _combine_last_body(s_ref, agg_ref, invd_ref, out_ref):
    agg = agg_ref[0] + agg_ref[1]
    out_ref[...] = jax.nn.sigmoid(s_ref[...] + agg * invd_ref[...])


def _combine_last(s_in, aggp, invd):
    return pl.pallas_call(
        _combine_last_body,
        grid=_GRID,
        in_specs=[_ROW, _PART, _ROW1],
        out_specs=_ROW,
        out_shape=jax.ShapeDtypeStruct((N, F), jnp.float32),
    )(s_in, aggp, invd)


_make_sc_pass = functools.lru_cache(maxsize=None)(_make_sc_pass)


def kernel(x, edge_index, W_self1, W_neigh1, b1, W_self2, W_neigh2, b2,
           W_self3, W_neigh3, b3):
    npad = E_PAD - E
    src = jnp.concatenate(
        [edge_index[0].astype(jnp.int32),
         jnp.zeros((npad,), jnp.int32)]).reshape(NCHUNKS_PAD, CHUNK)
    dst = jnp.concatenate(
        [edge_index[1].astype(jnp.int32),
         jnp.full((npad,), N, jnp.int32)]).reshape(NCHUNKS_PAD, CHUNK)
    zeros_f = jnp.zeros((SUB_ROWS, F), jnp.float32)
    zeros_1 = jnp.zeros((SUB_ROWS,), jnp.float32)
    b1r = b1.reshape(1, F)
    b2r = b2.reshape(1, F)
    b3r = b3.reshape(1, F)

    sc_pass_deg = _make_sc_pass(True)
    sc_pass = _make_sc_pass(False)

    p1, s1 = _prep(x, W_neigh1, W_self1, b1r)
    aggp1, degp = sc_pass_deg(src, dst, p1, zeros_f, zeros_1)
    deg0 = degp[:N].reshape(N, 1)
    deg1 = degp[N_PAD:N_PAD + N].reshape(N, 1)
    p2, s2, invd = _combine1(s1, aggp1[:, :N], deg0, deg1,
                             W_neigh2, W_self2, b2r)
    aggp2 = sc_pass(src, dst, p2, zeros_f)
    p3, s3 = _combine_mid(s2, aggp2[:, :N], invd, W_neigh3, W_self3, b3r)
    aggp3 = sc_pass(src, dst, p3, zeros_f)
    return _combine_last(s3, aggp3[:, :N], invd)


# speedup vs baseline: 2.9089x; 2.9089x over previous
import functools

import jax
import jax.numpy as jnp
from jax import lax
from jax.experimental import pallas as pl
from jax.experimental.pallas import tpu as pltpu
from jax.experimental.pallas import tpu_sc as plsc

N = 10000
E = 320000
F = 128

NC = 2
NS = 16
NW = NC * NS

CHUNK = 128
NCHUNKS_PAD = 2560
E_PAD = NCHUNKS_PAD * CHUNK
CH_W = NCHUNKS_PAD // NW
IB = 40
NIB = CH_W // IB
NJUNK = 128
N_PAD = N + NJUNK
SUB_ROWS = 640

_PREC = lax.Precision.HIGHEST


NBUF = 2


def _sc_body(with_deg, *refs):
    if with_deg:
        (src_hbm, dst_hbm, p_hbm, zf_hbm, z1_hbm,
         agg_hbm, deg_hbm, src_ib, dst_ib,
         rv0, rv1, ones_v, dbuf_v,
         agg_sh, deg_sh,
         g0, g1, s0, s1, dsem) = refs
    else:
        (src_hbm, dst_hbm, p_hbm, zf_hbm,
         agg_hbm, src_ib, dst_ib,
         rv0, rv1, agg_sh,
         g0, g1, s0, s1) = refs
    bufs = (rv0, rv1)
    gsems = (g0, g1)
    ssems = (s0, s1)

    c = lax.axis_index("c")
    s = lax.axis_index("s")
    w = s * NC + c
    rbase = pl.multiple_of(
        jnp.where(s < NS - 1, s * SUB_ROWS, N_PAD - SUB_ROWS), 8)

    pltpu.sync_copy(zf_hbm, agg_sh.at[pl.ds(rbase, SUB_ROWS)])
    if with_deg:
        pltpu.sync_copy(z1_hbm, dbuf_v)
        pltpu.sync_copy(dbuf_v, deg_sh.at[pl.ds(rbase, SUB_ROWS)])

        @pl.loop(0, CHUNK // 16)
        def _(i):
            ones_v[pl.ds(i * 16, 16)] = jnp.ones((16,), jnp.float32)
    plsc.subcore_barrier()

    def gather(i, b):
        pltpu.async_copy(p_hbm.at[src_ib.at[i]], bufs[b], gsems[b])

    def wait_gather(i, b):
        pltpu.make_async_copy(p_hbm.at[src_ib.at[i]], bufs[b], gsems[b]).wait()

    def scatter(i, b):
        pltpu.async_copy(bufs[b], agg_sh.at[dst_ib.at[i]], ssems[b], add=True)
        if with_deg:
            pltpu.async_copy(ones_v, deg_sh.at[dst_ib.at[i]], dsem, add=True)

    def wait_scatter(i, b):
        pltpu.make_async_copy(bufs[b], agg_sh.at[dst_ib.at[i]],
                              ssems[b]).wait()

    @pl.loop(0, NIB)
    def _(blk):
        crow = pl.multiple_of((w * NIB + blk) * IB, 8)
        pltpu.sync_copy(src_hbm.at[pl.ds(crow, IB)], src_ib)
        pltpu.sync_copy(dst_hbm.at[pl.ds(crow, IB)], dst_ib)
        gather(0, 0)
        for j in range(IB):
            if j + 1 < IB:
                if j >= 1:
                    wait_scatter(j - 1, (j + 1) % NBUF)
                gather(j + 1, (j + 1) % NBUF)
            wait_gather(j, j % NBUF)
            scatter(j, j % NBUF)
        wait_scatter(IB - 1, (IB - 1) % NBUF)
        if with_deg:
            for j in range(IB):
                pltpu.make_async_copy(ones_v, deg_sh.at[dst_ib.at[j]],
                                      dsem).wait()

    plsc.subcore_barrier()

    pltpu.sync_copy(agg_sh.at[pl.ds(rbase, SUB_ROWS)],
                    agg_hbm.at[c, pl.ds(rbase, SUB_ROWS)])
    if with_deg:
        dbase = pl.multiple_of(c * N_PAD + rbase, 8)
        pltpu.sync_copy(deg_sh.at[pl.ds(rbase, SUB_ROWS)], dbuf_v)
        pltpu.sync_copy(dbuf_v, deg_hbm.at[pl.ds(dbase, SUB_ROWS)])


def _make_sc_pass(with_deg):
    mesh = plsc.VectorSubcoreMesh(core_axis_name="c", subcore_axis_name="s",
                                  num_cores=NC, num_subcores=NS)
    out_type = [jax.ShapeDtypeStruct((NC, N_PAD, F), jnp.float32)]
    scratch = [
        pltpu.VMEM((IB, CHUNK), jnp.int32),
        pltpu.VMEM((IB, CHUNK), jnp.int32),
    ]
    scratch += [pltpu.VMEM((CHUNK, F), jnp.float32)] * NBUF

    if with_deg:
        out_type.append(jax.ShapeDtypeStruct((NC * N_PAD,), jnp.float32))
        scratch.append(pltpu.VMEM((CHUNK,), jnp.float32))
        scratch.append(pltpu.VMEM((SUB_ROWS,), jnp.float32))
    scratch.append(pltpu.VMEM_SHARED((N_PAD, F), jnp.float32))
    if with_deg:
        scratch.append(pltpu.VMEM_SHARED((N_PAD,), jnp.float32))
    scratch += [pltpu.SemaphoreType.DMA] * (2 * NBUF + (1 if with_deg else 0))

    return pl.kernel(functools.partial(_sc_body, with_deg),
                     out_type=tuple(out_type) if with_deg else out_type[0],
                     mesh=mesh, scratch_types=scratch)


BM = 1000

_ROW = pl.BlockSpec((BM, F), lambda i: (i, 0))
_ROW1 = pl.BlockSpec((BM, 1), lambda i: (i, 0))
_PART = pl.BlockSpec((NC, BM, F), lambda i: (0, i, 0))
_W = pl.BlockSpec((F, F), lambda i: (0, 0))
_B = pl.BlockSpec((1, F), lambda i: (0, 0))
_GRID = (N // BM,)


def _prep_body(x_ref, wn_ref, ws_ref, b_ref, p_ref, s_ref):
    x = x_ref[...]
    p_ref[...] = lax.dot(x, wn_ref[...], precision=_PREC)
    s_ref[...] = lax.dot(x, ws_ref[...], precision=_PREC) + b_ref[...]


def _prep(x, wn, ws, b):
    return pl.pallas_call(
        _prep_body,
        grid=_GRID,
        in_specs=[_ROW, _W, _W, _B],
        out_specs=(_ROW, _ROW),
        out_shape=(jax.ShapeDtypeStruct((N, F), jnp.float32),
                   jax.ShapeDtypeStruct((N, F), jnp.float32)),
    )(x, wn, ws, b)


def _combine1_body(s_ref, agg_ref, deg0_ref, deg1_ref, wn_ref, ws_ref, b_ref,
                   p_ref, s2_ref, invd_ref):
    deg = deg0_ref[...] + deg1_ref[...]
    invd = 1.0 / jnp.maximum(deg, 1.0)
    invd_ref[...] = invd
    agg = agg_ref[0] + agg_ref[1]
    h = jax.nn.sigmoid(s_ref[...] + agg * invd)
    p_ref[...] = lax.dot(h, wn_ref[...], precision=_PREC)
    s2_ref[...] = lax.dot(h, ws_ref[...], precision=_PREC) + b_ref[...]


def _combine1(s1, aggp, deg0, deg1, wn, ws, b):
    return pl.pallas_call(
        _combine1_body,
        grid=_GRID,
        in_specs=[_ROW, _PART, _ROW1, _ROW1, _W, _W, _B],
        out_specs=(_ROW, _ROW, _ROW1),
        out_shape=(jax.ShapeDtypeStruct((N, F), jnp.float32),
                   jax.ShapeDtypeStruct((N, F), jnp.float32),
                   jax.ShapeDtypeStruct((N, 1), jnp.float32)),
    )(s1, aggp, deg0, deg1, wn, ws, b)


def _combine_mid_body(s_ref, agg_ref, invd_ref, wn_ref, ws_ref, b_ref,
                      p_ref, s2_ref):
    agg = agg_ref[0] + agg_ref[1]
    h = jax.nn.sigmoid(s_ref[...] + agg * invd_ref[...])
    p_ref[...] = lax.dot(h, wn_ref[...], precision=_PREC)
    s2_ref[...] = lax.dot(h, ws_ref[...], precision=_PREC) + b_ref[...]


def _combine_mid(s_in, aggp, invd, wn, ws, b):
    return pl.pallas_call(
        _combine_mid_body,
        grid=_GRID,
        in_specs=[_ROW, _PART, _ROW1, _W, _W, _B],
        out_specs=(_ROW, _ROW),
        out_shape=(jax.ShapeDtypeStruct((N, F), jnp.float32),
                   jax.ShapeDtypeStruct((N, F), jnp.float32)),
    )(s_in, aggp, invd, wn, ws, b)


def _combine_last_body(s_ref, agg_ref, invd_ref, out_ref):
    agg = agg_ref[0] + agg_ref[1]
    out_ref[...] = jax.nn.sigmoid(s_ref[...] + agg * invd_ref[...])


def _combine_last(s_in, aggp, invd):
    return pl.pallas_call(
        _combine_last_body,
        grid=_GRID,
        in_specs=[_ROW, _PART, _ROW1],
        out_specs=_ROW,
        out_shape=jax.ShapeDtypeStruct((N, F), jnp.float32),
    )(s_in, aggp, invd)


_make_sc_pass = functools.lru_cache(maxsize=None)(_make_sc_pass)


def kernel(x, edge_index, W_self1, W_neigh1, b1, W_self2, W_neigh2, b2,
           W_self3, W_neigh3, b3):
    npad = E_PAD - E
    pad_iota = jnp.arange(npad, dtype=jnp.int32)
    src = jnp.concatenate(
        [edge_index[0].astype(jnp.int32),
         pad_iota % N]).reshape(NCHUNKS_PAD, CHUNK)
    dst = jnp.concatenate(
        [edge_index[1].astype(jnp.int32),
         N + pad_iota % NJUNK]).reshape(NCHUNKS_PAD, CHUNK)
    zeros_f = jnp.zeros((SUB_ROWS, F), jnp.float32)
    zeros_1 = jnp.zeros((SUB_ROWS,), jnp.float32)
    b1r = b1.reshape(1, F)
    b2r = b2.reshape(1, F)
    b3r = b3.reshape(1, F)

    sc_pass_deg = _make_sc_pass(True)
    sc_pass = _make_sc_pass(False)

    p1, s1 = _prep(x, W_neigh1, W_self1, b1r)
    aggp1, degp = sc_pass_deg(src, dst, p1, zeros_f, zeros_1)
    deg0 = degp[:N].reshape(N, 1)
    deg1 = degp[N_PAD:N_PAD + N].reshape(N, 1)
    p2, s2, invd = _combine1(s1, aggp1[:, :N], deg0, deg1,
                             W_neigh2, W_self2, b2r)
    aggp2 = sc_pass(src, dst, p2, zeros_f)
    p3, s3 = _combine_mid(s2, aggp2[:, :N], invd, W_neigh3, W_self3, b3r)
    aggp3 = sc_pass(src, dst, p3, zeros_f)
    return _combine_last(s3, aggp3[:, :N], invd)
